# baseline (speedup 1.0000x reference)
import functools

import jax
import jax.numpy as jnp
from jax import lax
from jax.experimental import pallas as pl
from jax.experimental.pallas import tpu as pltpu
from jax.experimental.pallas import tpu_sc as plsc

_L = 16
_NC = 2
_NS = 16
_IDX_CHUNK = 128
_BL = 24576


@functools.lru_cache(maxsize=None)
def _build_tc_scores(V, D, bl):
    nb = (V + bl - 1) // bl

    def body(tu_ref, ti_ref, w_ref, ou_ref, oi_ref):
        w = w_ref[...]
        pu = jax.lax.dot_general(
            w[:D], tu_ref[...], (((0,), (0,)), ((), ())),
            preferred_element_type=jnp.float32)
        ou_ref[...] = pu.reshape(ou_ref.shape)
        pi = jax.lax.dot_general(
            w[D:], ti_ref[...], (((0,), (0,)), ((), ())),
            preferred_element_type=jnp.float32)
        oi_ref[...] = pi.reshape(oi_ref.shape)

    return pl.pallas_call(
        body,
        grid=(nb,),
        in_specs=[
            pl.BlockSpec((D, bl), lambda i: (0, i)),
            pl.BlockSpec((D, bl), lambda i: (0, i)),
            pl.BlockSpec((2 * D, 1), lambda i: (0, 0)),
        ],
        out_specs=[
            pl.BlockSpec((bl,), lambda i: (i,)),
            pl.BlockSpec((bl,), lambda i: (i,)),
        ],
        out_shape=[
            jax.ShapeDtypeStruct((V,), jnp.float32),
            jax.ShapeDtypeStruct((V,), jnp.float32),
        ],
        compiler_params=pltpu.CompilerParams(
            dimension_semantics=("parallel",)),
    )


@functools.lru_cache(maxsize=None)
def _build_sc_gather(B):
    nw = _NC * _NS
    bpw = B // nw
    n_chunk = bpw // _IDX_CHUNK
    n_grp = bpw // _L

    mesh = plsc.VectorSubcoreMesh(core_axis_name="c", subcore_axis_name="s")

    @functools.partial(
        pl.kernel,
        mesh=mesh,
        compiler_params=pltpu.CompilerParams(
            needs_layout_passes=False, use_tc_tiling_on_sc=False),
        out_type=jax.ShapeDtypeStruct((B,), jnp.float32),
        scratch_types=[
            pltpu.VMEM((n_chunk, _IDX_CHUNK), jnp.int32),
            pltpu.VMEM((n_chunk, _IDX_CHUNK), jnp.int32),
            pltpu.VMEM((bpw,), jnp.float32),
            pltpu.VMEM((bpw,), jnp.float32),
            pltpu.VMEM((_L,), jnp.float32),
            pltpu.VMEM((bpw,), jnp.float32),
            pltpu.SemaphoreType.DMA,
            pltpu.SemaphoreType.DMA,
        ],
    )
    def sc_kernel(uidx_hbm, iidx_hbm, su_hbm, si_hbm, b_hbm, out_hbm,
                  uidx_v, iidx_v, su_v, si_v, b_v, out_v, sem_u, sem_i):
        wid = lax.axis_index("s") * _NC + lax.axis_index("c")
        crow = wid * n_chunk

        pltpu.sync_copy(uidx_hbm.at[pl.ds(crow, n_chunk), :], uidx_v)
        pltpu.sync_copy(iidx_hbm.at[pl.ds(crow, n_chunk), :], iidx_v)
        pltpu.sync_copy(b_hbm, b_v)

        copies = []
        for j in range(n_chunk):
            dst = pl.ds(j * _IDX_CHUNK, _IDX_CHUNK)
            copies.append(pltpu.async_copy(
                su_hbm.at[uidx_v.at[j]], su_v.at[dst], sem_u))
            copies.append(pltpu.async_copy(
                si_hbm.at[iidx_v.at[j]], si_v.at[dst], sem_i))
        for cp in copies:
            cp.wait()

        bv = b_v[...]

        def grp_body(g, carry):
            s = pl.multiple_of(g * _L, _L)
            x = su_v[pl.ds(s, _L)] + si_v[pl.ds(s, _L)] + bv
            out_v[pl.ds(s, _L)] = 1.0 / (1.0 + jnp.exp(-x))
            return carry

        lax.fori_loop(0, n_grp, grp_body, 0)

        pltpu.sync_copy(out_v, out_hbm.at[pl.ds(wid * bpw, bpw)])

    return sc_kernel


def kernel(user_input, pos_item_input, user_table, item_table, W, b):
    B = user_input.shape[0]
    V_u, D = user_table.shape
    V_i = item_table.shape[0]
    uidx = user_input.reshape(B // _IDX_CHUNK, _IDX_CHUNK).astype(jnp.int32)
    iidx = pos_item_input.reshape(B // _IDX_CHUNK, _IDX_CHUNK).astype(jnp.int32)
    b16 = jnp.broadcast_to(b.reshape(()), (_L,)).astype(jnp.float32)
    scores_u, scores_i = _build_tc_scores(V_u, D, _BL)(
        user_table.T, item_table.T, W.astype(jnp.float32))
    out = _build_sc_gather(B)(uidx, iidx, scores_u, scores_i, b16)
    return out.reshape(B, 1)


# speedup vs baseline: 1.0010x; 1.0010x over previous
import functools

import jax
import jax.numpy as jnp
from jax import lax
from jax.experimental import pallas as pl
from jax.experimental.pallas import tpu as pltpu
from jax.experimental.pallas import tpu_sc as plsc

_L = 16
_NC = 2
_NS = 16
_IDX_CHUNK = 128
_BL = 40960


@functools.lru_cache(maxsize=None)
def _build_tc_scores(V, D, bl):
    nb = (V + bl - 1) // bl

    def body(tu_ref, ti_ref, w_ref, ou_ref, oi_ref):
        w = w_ref[...]
        pu = jax.lax.dot_general(
            w[:D], tu_ref[...], (((0,), (0,)), ((), ())),
            preferred_element_type=jnp.float32)
        ou_ref[...] = pu.reshape(ou_ref.shape)
        pi = jax.lax.dot_general(
            w[D:], ti_ref[...], (((0,), (0,)), ((), ())),
            preferred_element_type=jnp.float32)
        oi_ref[...] = pi.reshape(oi_ref.shape)

    return pl.pallas_call(
        body,
        grid=(nb,),
        in_specs=[
            pl.BlockSpec((D, bl), lambda i: (0, i)),
            pl.BlockSpec((D, bl), lambda i: (0, i)),
            pl.BlockSpec((2 * D, 1), lambda i: (0, 0)),
        ],
        out_specs=[
            pl.BlockSpec((bl,), lambda i: (i,)),
            pl.BlockSpec((bl,), lambda i: (i,)),
        ],
        out_shape=[
            jax.ShapeDtypeStruct((V,), jnp.float32),
            jax.ShapeDtypeStruct((V,), jnp.float32),
        ],
        compiler_params=pltpu.CompilerParams(
            dimension_semantics=("parallel",)),
    )


@functools.lru_cache(maxsize=None)
def _build_sc_gather(B):
    nw = _NC * _NS
    bpw = B // nw
    n_chunk = bpw // _IDX_CHUNK
    n_grp = bpw // _L

    mesh = plsc.VectorSubcoreMesh(core_axis_name="c", subcore_axis_name="s")

    @functools.partial(
        pl.kernel,
        mesh=mesh,
        compiler_params=pltpu.CompilerParams(
            needs_layout_passes=False, use_tc_tiling_on_sc=False),
        out_type=jax.ShapeDtypeStruct((B,), jnp.float32),
        scratch_types=[
            pltpu.VMEM((n_chunk, _IDX_CHUNK), jnp.int32),
            pltpu.VMEM((n_chunk, _IDX_CHUNK), jnp.int32),
            pltpu.VMEM((bpw,), jnp.float32),
            pltpu.VMEM((bpw,), jnp.float32),
            pltpu.VMEM((_L,), jnp.float32),
            pltpu.VMEM((bpw,), jnp.float32),
            pltpu.SemaphoreType.DMA,
            pltpu.SemaphoreType.DMA,
        ],
    )
    def sc_kernel(uidx_hbm, iidx_hbm, su_hbm, si_hbm, b_hbm, out_hbm,
                  uidx_v, iidx_v, su_v, si_v, b_v, out_v, sem_u, sem_i):
        wid = lax.axis_index("s") * _NC + lax.axis_index("c")
        crow = wid * n_chunk

        pltpu.sync_copy(uidx_hbm.at[pl.ds(crow, n_chunk), :], uidx_v)
        pltpu.sync_copy(iidx_hbm.at[pl.ds(crow, n_chunk), :], iidx_v)
        pltpu.sync_copy(b_hbm, b_v)

        copies = []
        for j in range(n_chunk):
            dst = pl.ds(j * _IDX_CHUNK, _IDX_CHUNK)
            copies.append(pltpu.async_copy(
                su_hbm.at[uidx_v.at[j]], su_v.at[dst], sem_u))
            copies.append(pltpu.async_copy(
                si_hbm.at[iidx_v.at[j]], si_v.at[dst], sem_i))
        for cp in copies:
            cp.wait()

        bv = b_v[...]

        def grp_body(g, carry):
            s = pl.multiple_of(g * _L, _L)
            x = su_v[pl.ds(s, _L)] + si_v[pl.ds(s, _L)] + bv
            out_v[pl.ds(s, _L)] = 1.0 / (1.0 + jnp.exp(-x))
            return carry

        lax.fori_loop(0, n_grp, grp_body, 0)

        pltpu.sync_copy(out_v, out_hbm.at[pl.ds(wid * bpw, bpw)])

    return sc_kernel


def kernel(user_input, pos_item_input, user_table, item_table, W, b):
    B = user_input.shape[0]
    V_u, D = user_table.shape
    V_i = item_table.shape[0]
    uidx = user_input.reshape(B // _IDX_CHUNK, _IDX_CHUNK).astype(jnp.int32)
    iidx = pos_item_input.reshape(B // _IDX_CHUNK, _IDX_CHUNK).astype(jnp.int32)
    b16 = jnp.broadcast_to(b.reshape(()), (_L,)).astype(jnp.float32)
    scores_u, scores_i = _build_tc_scores(V_u, D, _BL)(
        user_table.T, item_table.T, W.astype(jnp.float32))
    out = _build_sc_gather(B)(uidx, iidx, scores_u, scores_i, b16)
    return out.reshape(B, 1)


# speedup vs baseline: 1.0069x; 1.0058x over previous
import functools

import jax
import jax.numpy as jnp
from jax import lax
from jax.experimental import pallas as pl
from jax.experimental.pallas import tpu as pltpu
from jax.experimental.pallas import tpu_sc as plsc

_L = 16
_NC = 2
_NS = 16
_IDX_CHUNK = 128
_BL = 32768


@functools.lru_cache(maxsize=None)
def _build_tc_scores(V, D, bl):
    nb = (V + bl - 1) // bl

    def body(tu_ref, ti_ref, w_ref, ou_ref, oi_ref):
        w = w_ref[...]
        pu = jax.lax.dot_general(
            w[:D], tu_ref[...], (((0,), (0,)), ((), ())),
            preferred_element_type=jnp.float32)
        ou_ref[...] = pu.reshape(ou_ref.shape)
        pi = jax.lax.dot_general(
            w[D:], ti_ref[...], (((0,), (0,)), ((), ())),
            preferred_element_type=jnp.float32)
        oi_ref[...] = pi.reshape(oi_ref.shape)

    return pl.pallas_call(
        body,
        grid=(nb,),
        in_specs=[
            pl.BlockSpec((D, bl), lambda i: (0, i)),
            pl.BlockSpec((D, bl), lambda i: (0, i)),
            pl.BlockSpec((2 * D, 1), lambda i: (0, 0)),
        ],
        out_specs=[
            pl.BlockSpec((bl,), lambda i: (i,)),
            pl.BlockSpec((bl,), lambda i: (i,)),
        ],
        out_shape=[
            jax.ShapeDtypeStruct((V,), jnp.float32),
            jax.ShapeDtypeStruct((V,), jnp.float32),
        ],
        compiler_params=pltpu.CompilerParams(
            dimension_semantics=("parallel",)),
    )


@functools.lru_cache(maxsize=None)
def _build_sc_gather(B):
    nw = _NC * _NS
    bpw = B // nw
    n_chunk = bpw // _IDX_CHUNK
    n_grp = bpw // _L

    mesh = plsc.VectorSubcoreMesh(core_axis_name="c", subcore_axis_name="s")

    @functools.partial(
        pl.kernel,
        mesh=mesh,
        compiler_params=pltpu.CompilerParams(
            needs_layout_passes=False, use_tc_tiling_on_sc=False),
        out_type=jax.ShapeDtypeStruct((B,), jnp.float32),
        scratch_types=[
            pltpu.VMEM((n_chunk, _IDX_CHUNK), jnp.int32),
            pltpu.VMEM((n_chunk, _IDX_CHUNK), jnp.int32),
            pltpu.VMEM((bpw,), jnp.float32),
            pltpu.VMEM((bpw,), jnp.float32),
            pltpu.VMEM((_L,), jnp.float32),
            pltpu.VMEM((bpw,), jnp.float32),
            pltpu.SemaphoreType.DMA,
            pltpu.SemaphoreType.DMA,
        ],
    )
    def sc_kernel(uidx_hbm, iidx_hbm, su_hbm, si_hbm, b_hbm, out_hbm,
                  uidx_v, iidx_v, su_v, si_v, b_v, out_v, sem_u, sem_i):
        wid = lax.axis_index("s") * _NC + lax.axis_index("c")
        crow = wid * n_chunk

        pltpu.sync_copy(uidx_hbm.at[pl.ds(crow, n_chunk), :], uidx_v)
        pltpu.sync_copy(iidx_hbm.at[pl.ds(crow, n_chunk), :], iidx_v)
        pltpu.sync_copy(b_hbm, b_v)

        copies = []
        for j in range(n_chunk):
            dst = pl.ds(j * _IDX_CHUNK, _IDX_CHUNK)
            copies.append(pltpu.async_copy(
                su_hbm.at[uidx_v.at[j]], su_v.at[dst], sem_u))
            copies.append(pltpu.async_copy(
                si_hbm.at[iidx_v.at[j]], si_v.at[dst], sem_i))
        for cp in copies:
            cp.wait()

        bv = b_v[...]

        def grp_body(g, carry):
            s = pl.multiple_of(g * _L, _L)
            x = su_v[pl.ds(s, _L)] + si_v[pl.ds(s, _L)] + bv
            out_v[pl.ds(s, _L)] = 1.0 / (1.0 + jnp.exp(-x))
            return carry

        lax.fori_loop(0, n_grp, grp_body, 0)

        pltpu.sync_copy(out_v, out_hbm.at[pl.ds(wid * bpw, bpw)])

    return sc_kernel


def kernel(user_input, pos_item_input, user_table, item_table, W, b):
    B = user_input.shape[0]
    V_u, D = user_table.shape
    V_i = item_table.shape[0]
    uidx = user_input.reshape(B // _IDX_CHUNK, _IDX_CHUNK).astype(jnp.int32)
    iidx = pos_item_input.reshape(B // _IDX_CHUNK, _IDX_CHUNK).astype(jnp.int32)
    b16 = jnp.broadcast_to(b.reshape(()), (_L,)).astype(jnp.float32)
    scores_u, scores_i = _build_tc_scores(V_u, D, _BL)(
        user_table.T, item_table.T, W.astype(jnp.float32))
    out = _build_sc_gather(B)(uidx, iidx, scores_u, scores_i, b16)
    return out.reshape(B, 1)
